# baseline (speedup 1.0000x reference)
import jax
import jax.numpy as jnp
from jax.experimental import pallas as pl

_F32 = jnp.float32


def _dot(a, b):
    return jnp.dot(a, b, preferred_element_type=_F32)


def _dot_t(a, b):
    return jax.lax.dot_general(a, b, (((0,), (0,)), ((), ())),
                               preferred_element_type=_F32)


def _k1(h_ref, x_ref, e0_ref, de_ref):
    @pl.when(pl.program_id(0) == 0)
    def _():
        e0_ref[...] = jnp.zeros_like(e0_ref)
        de_ref[...] = jnp.zeros_like(de_ref)

    h = h_ref[...]
    e0_ref[...] += _dot_t(h, x_ref[...])
    de_ref[...] += jnp.sum(h, axis=0, keepdims=True)


def _k2(h_ref, g0_ref, b0_ref, x_ref, ep_ref):
    @pl.when(pl.program_id(0) == 0)
    def _():
        ep_ref[...] = jnp.zeros_like(ep_ref)

    h = h_ref[...]
    dv = jnp.clip(jnp.sum(h, axis=1, keepdims=True), 1e-6, None)
    xb = jnp.maximum(_dot(h, g0_ref[...]) / dv + b0_ref[...], 0.0)
    x_ref[...] = xb
    ep_ref[...] += _dot_t(h, xb)


def _k3(h_ref, ep_ref, wpt_ref, bp_ref, x_ref, s_ref, x2_ref, dv2_ref):
    @pl.when(pl.program_id(0) == 0)
    def _():
        x2_ref[...] = jnp.zeros_like(x2_ref)
        dv2_ref[...] = jnp.zeros_like(dv2_ref)

    h = h_ref[...]
    dv_raw = jnp.sum(h, axis=1, keepdims=True)
    dv = jnp.clip(dv_raw, 1e-6, None)
    t = _dot(h, ep_ref[...]) / dv
    logits = _dot(t, wpt_ref[...]) + bp_ref[...]
    mx = jnp.max(logits, axis=1, keepdims=True)
    ex = jnp.exp(logits - mx)
    s = ex / jnp.sum(ex, axis=1, keepdims=True)
    s_ref[...] = s
    x2_ref[...] += _dot_t(s, x_ref[...])
    dv2_ref[...] += _dot_t(s, dv_raw)


def _k4(h_ref, s_ref, x2_ref, e2_ref):
    @pl.when(pl.program_id(0) == 0)
    def _():
        e2_ref[...] = jnp.zeros_like(e2_ref)

    b = _dot(s_ref[...], x2_ref[...])
    e2_ref[...] += _dot_t(h_ref[...], b)


def _k5(h_ref, s_ref, a_ref, xn2_ref):
    @pl.when(pl.program_id(0) == 0)
    def _():
        xn2_ref[...] = jnp.zeros_like(xn2_ref)

    c = _dot(h_ref[...], a_ref[...])
    xn2_ref[...] += _dot_t(s_ref[...], c)


def _row_block(n):
    for bn in (1000, 400, 200, 80, 40, 16, 8):
        if n % bn == 0:
            return bn
    return n


def kernel(node_features, incidence, W0, b0, Wp, bp, W1, b1, Wr, br):
    n, m = incidence.shape
    d = node_features.shape[1]
    k = Wp.shape[0]
    bn = _row_block(n)
    nb = n // bn
    grid = (nb,)

    def row_spec(cols):
        return pl.BlockSpec((bn, cols), lambda i: (i, 0))

    def acc_spec(rows, cols):
        return pl.BlockSpec((rows, cols), lambda i: (0, 0))

    f32 = jnp.float32

    e0_raw, de_row = pl.pallas_call(
        _k1,
        grid=grid,
        in_specs=[row_spec(m), row_spec(d)],
        out_specs=[acc_spec(m, d), acc_spec(1, m)],
        out_shape=[jax.ShapeDtypeStruct((m, d), f32),
                   jax.ShapeDtypeStruct((1, m), f32)],
    )(incidence, node_features)

    de = jnp.clip(de_row[0], 1e-6, None)
    g0 = _dot(e0_raw / de[:, None], W0.T)

    x, ep_raw = pl.pallas_call(
        _k2,
        grid=grid,
        in_specs=[row_spec(m), acc_spec(m, d), acc_spec(1, d)],
        out_specs=[row_spec(d), acc_spec(m, d)],
        out_shape=[jax.ShapeDtypeStruct((n, d), f32),
                   jax.ShapeDtypeStruct((m, d), f32)],
    )(incidence, g0, b0[None, :])

    ep = ep_raw / de[:, None]

    s, x2, dv2_col = pl.pallas_call(
        _k3,
        grid=grid,
        in_specs=[row_spec(m), acc_spec(m, d), acc_spec(d, k),
                  acc_spec(1, k), row_spec(d)],
        out_specs=[row_spec(k), acc_spec(k, d), acc_spec(k, 1)],
        out_shape=[jax.ShapeDtypeStruct((n, k), f32),
                   jax.ShapeDtypeStruct((k, d), f32),
                   jax.ShapeDtypeStruct((k, 1), f32)],
    )(incidence, ep, Wp.T, bp[None, :], x)

    e2_raw = pl.pallas_call(
        _k4,
        grid=grid,
        in_specs=[row_spec(m), row_spec(k), acc_spec(k, d)],
        out_specs=acc_spec(m, d),
        out_shape=jax.ShapeDtypeStruct((m, d), f32),
    )(incidence, s, x2)

    a = _dot(e2_raw / de[:, None], W1.T)

    xn2 = pl.pallas_call(
        _k5,
        grid=grid,
        in_specs=[row_spec(m), row_spec(k), acc_spec(m, d)],
        out_specs=acc_spec(k, d),
        out_shape=jax.ShapeDtypeStruct((k, d), f32),
    )(incidence, s, a)

    dv2 = jnp.clip(dv2_col[:, 0], 1e-6, None)
    x3 = jnp.maximum(xn2 / dv2[:, None] + b1[None, :], 0.0)
    return _dot(Wr, jnp.mean(x3, axis=0)) + br


# speedup vs baseline: 1.0118x; 1.0118x over previous
import jax
import jax.numpy as jnp
from jax.experimental import pallas as pl

_F32 = jnp.float32
_BF16 = jnp.bfloat16


def _dot(a, b):
    return jnp.dot(a.astype(_BF16), b.astype(_BF16),
                   preferred_element_type=_F32)


def _dot_t(a, b):
    return jax.lax.dot_general(a.astype(_BF16), b.astype(_BF16),
                               (((0,), (0,)), ((), ())),
                               preferred_element_type=_F32)


def _k1(h_ref, x_ref, e0_ref, de_ref):
    @pl.when(pl.program_id(0) == 0)
    def _():
        e0_ref[...] = jnp.zeros_like(e0_ref)
        de_ref[...] = jnp.zeros_like(de_ref)

    h = h_ref[...]
    e0_ref[...] += _dot_t(h, x_ref[...])
    de_ref[...] += jnp.sum(h, axis=0, keepdims=True)


def _k2(h_ref, g0_ref, b0_ref, x_ref, ep_ref):
    @pl.when(pl.program_id(0) == 0)
    def _():
        ep_ref[...] = jnp.zeros_like(ep_ref)

    h = h_ref[...]
    dv = jnp.clip(jnp.sum(h, axis=1, keepdims=True), 1e-6, None)
    xb = jnp.maximum(_dot(h, g0_ref[...]) / dv + b0_ref[...], 0.0)
    x_ref[...] = xb
    ep_ref[...] += _dot_t(h, xb)


def _k3(h_ref, ep_ref, wpt_ref, bp_ref, x_ref, s_ref, x2_ref, dv2_ref):
    @pl.when(pl.program_id(0) == 0)
    def _():
        x2_ref[...] = jnp.zeros_like(x2_ref)
        dv2_ref[...] = jnp.zeros_like(dv2_ref)

    h = h_ref[...]
    dv_raw = jnp.sum(h, axis=1, keepdims=True)
    dv = jnp.clip(dv_raw, 1e-6, None)
    t = _dot(h, ep_ref[...]) / dv
    logits = _dot(t, wpt_ref[...]) + bp_ref[...]
    mx = jnp.max(logits, axis=1, keepdims=True)
    ex = jnp.exp(logits - mx)
    s = ex / jnp.sum(ex, axis=1, keepdims=True)
    s_ref[...] = s
    x2_ref[...] += _dot_t(s, x_ref[...])
    dv2_ref[...] += _dot_t(s, dv_raw)


def _k4(h_ref, s_ref, x2_ref, e2_ref):
    @pl.when(pl.program_id(0) == 0)
    def _():
        e2_ref[...] = jnp.zeros_like(e2_ref)

    b = _dot(s_ref[...], x2_ref[...])
    e2_ref[...] += _dot_t(h_ref[...], b)


def _k5(h_ref, s_ref, a_ref, xn2_ref):
    @pl.when(pl.program_id(0) == 0)
    def _():
        xn2_ref[...] = jnp.zeros_like(xn2_ref)

    c = _dot(h_ref[...], a_ref[...])
    xn2_ref[...] += _dot_t(s_ref[...], c)


def _row_block(n):
    for bn in (1000, 400, 200, 80, 40, 16, 8):
        if n % bn == 0:
            return bn
    return n


def kernel(node_features, incidence, W0, b0, Wp, bp, W1, b1, Wr, br):
    n, m = incidence.shape
    d = node_features.shape[1]
    k = Wp.shape[0]
    bn = _row_block(n)
    nb = n // bn
    grid = (nb,)

    def row_spec(cols):
        return pl.BlockSpec((bn, cols), lambda i: (i, 0))

    def acc_spec(rows, cols):
        return pl.BlockSpec((rows, cols), lambda i: (0, 0))

    f32 = jnp.float32

    e0_raw, de_row = pl.pallas_call(
        _k1,
        grid=grid,
        in_specs=[row_spec(m), row_spec(d)],
        out_specs=[acc_spec(m, d), acc_spec(1, m)],
        out_shape=[jax.ShapeDtypeStruct((m, d), f32),
                   jax.ShapeDtypeStruct((1, m), f32)],
    )(incidence, node_features)

    de = jnp.clip(de_row[0], 1e-6, None)
    g0 = _dot(e0_raw / de[:, None], W0.T)

    x, ep_raw = pl.pallas_call(
        _k2,
        grid=grid,
        in_specs=[row_spec(m), acc_spec(m, d), acc_spec(1, d)],
        out_specs=[row_spec(d), acc_spec(m, d)],
        out_shape=[jax.ShapeDtypeStruct((n, d), f32),
                   jax.ShapeDtypeStruct((m, d), f32)],
    )(incidence, g0, b0[None, :])

    ep = ep_raw / de[:, None]

    s, x2, dv2_col = pl.pallas_call(
        _k3,
        grid=grid,
        in_specs=[row_spec(m), acc_spec(m, d), acc_spec(d, k),
                  acc_spec(1, k), row_spec(d)],
        out_specs=[row_spec(k), acc_spec(k, d), acc_spec(k, 1)],
        out_shape=[jax.ShapeDtypeStruct((n, k), f32),
                   jax.ShapeDtypeStruct((k, d), f32),
                   jax.ShapeDtypeStruct((k, 1), f32)],
    )(incidence, ep, Wp.T, bp[None, :], x)

    e2_raw = pl.pallas_call(
        _k4,
        grid=grid,
        in_specs=[row_spec(m), row_spec(k), acc_spec(k, d)],
        out_specs=acc_spec(m, d),
        out_shape=jax.ShapeDtypeStruct((m, d), f32),
    )(incidence, s, x2)

    a = _dot(e2_raw / de[:, None], W1.T)

    xn2 = pl.pallas_call(
        _k5,
        grid=grid,
        in_specs=[row_spec(m), row_spec(k), acc_spec(m, d)],
        out_specs=acc_spec(k, d),
        out_shape=jax.ShapeDtypeStruct((k, d), f32),
    )(incidence, s, a)

    dv2 = jnp.clip(dv2_col[:, 0], 1e-6, None)
    x3 = jnp.maximum(xn2 / dv2[:, None] + b1[None, :], 0.0)
    return _dot(Wr, jnp.mean(x3, axis=0)) + br


# speedup vs baseline: 1.0672x; 1.0548x over previous
import jax
import jax.numpy as jnp
from jax.experimental import pallas as pl

_F32 = jnp.float32
_BF16 = jnp.bfloat16


def _dot(a, b):
    return jnp.dot(a.astype(_BF16), b.astype(_BF16),
                   preferred_element_type=_F32)


def _dot_t(a, b):
    return jax.lax.dot_general(a.astype(_BF16), b.astype(_BF16),
                               (((0,), (0,)), ((), ())),
                               preferred_element_type=_F32)


def _k1(h_ref, x_ref, hb_ref, e0_ref, de_ref):
    @pl.when(pl.program_id(0) == 0)
    def _():
        e0_ref[...] = jnp.zeros_like(e0_ref)
        de_ref[...] = jnp.zeros_like(de_ref)

    h = h_ref[...]
    hb_ref[...] = h.astype(_BF16)
    e0_ref[...] += _dot_t(h, x_ref[...])
    de_ref[...] += jnp.sum(h, axis=0, keepdims=True)


def _k2(h_ref, g0_ref, b0_ref, x_ref, ep_ref):
    @pl.when(pl.program_id(0) == 0)
    def _():
        ep_ref[...] = jnp.zeros_like(ep_ref)

    h = h_ref[...]
    dv = jnp.clip(jnp.sum(h, axis=1, keepdims=True, dtype=_F32), 1e-6, None)
    xb = jnp.maximum(_dot(h, g0_ref[...]) / dv + b0_ref[...], 0.0)
    x_ref[...] = xb.astype(_BF16)
    ep_ref[...] += _dot_t(h, xb)


def _k3(h_ref, ep_ref, wpt_ref, bp_ref, x_ref, s_ref, x2_ref, dv2_ref):
    @pl.when(pl.program_id(0) == 0)
    def _():
        x2_ref[...] = jnp.zeros_like(x2_ref)
        dv2_ref[...] = jnp.zeros_like(dv2_ref)

    h = h_ref[...]
    dv_raw = jnp.sum(h, axis=1, keepdims=True, dtype=_F32)
    dv = jnp.clip(dv_raw, 1e-6, None)
    t = _dot(h, ep_ref[...]) / dv
    logits = _dot(t, wpt_ref[...]) + bp_ref[...]
    mx = jnp.max(logits, axis=1, keepdims=True)
    ex = jnp.exp(logits - mx)
    s = ex / jnp.sum(ex, axis=1, keepdims=True)
    s_ref[...] = s.astype(_BF16)
    x2_ref[...] += _dot_t(s, x_ref[...])
    dv2_ref[...] += _dot_t(s, dv_raw)


def _k4(h_ref, s_ref, x2_ref, e2_ref):
    @pl.when(pl.program_id(0) == 0)
    def _():
        e2_ref[...] = jnp.zeros_like(e2_ref)

    b = _dot(s_ref[...], x2_ref[...])
    e2_ref[...] += _dot_t(h_ref[...], b)


def _k5(h_ref, s_ref, a_ref, xn2_ref):
    @pl.when(pl.program_id(0) == 0)
    def _():
        xn2_ref[...] = jnp.zeros_like(xn2_ref)

    c = _dot(h_ref[...], a_ref[...])
    xn2_ref[...] += _dot_t(s_ref[...], c)


def _row_block(n):
    for bn in (1000, 400, 200, 80, 40, 16, 8):
        if n % bn == 0:
            return bn
    return n


def kernel(node_features, incidence, W0, b0, Wp, bp, W1, b1, Wr, br):
    n, m = incidence.shape
    d = node_features.shape[1]
    k = Wp.shape[0]
    bn = _row_block(n)
    nb = n // bn
    grid = (nb,)

    def row_spec(cols):
        return pl.BlockSpec((bn, cols), lambda i: (i, 0))

    def acc_spec(rows, cols):
        return pl.BlockSpec((rows, cols), lambda i: (0, 0))

    f32 = jnp.float32
    bf16 = jnp.bfloat16

    hb, e0_raw, de_row = pl.pallas_call(
        _k1,
        grid=grid,
        in_specs=[row_spec(m), row_spec(d)],
        out_specs=[row_spec(m), acc_spec(m, d), acc_spec(1, m)],
        out_shape=[jax.ShapeDtypeStruct((n, m), bf16),
                   jax.ShapeDtypeStruct((m, d), f32),
                   jax.ShapeDtypeStruct((1, m), f32)],
    )(incidence, node_features)

    de = jnp.clip(de_row[0], 1e-6, None)
    g0 = _dot(e0_raw / de[:, None], W0.T)

    x, ep_raw = pl.pallas_call(
        _k2,
        grid=grid,
        in_specs=[row_spec(m), acc_spec(m, d), acc_spec(1, d)],
        out_specs=[row_spec(d), acc_spec(m, d)],
        out_shape=[jax.ShapeDtypeStruct((n, d), bf16),
                   jax.ShapeDtypeStruct((m, d), f32)],
    )(hb, g0, b0[None, :])

    ep = (ep_raw / de[:, None]).astype(bf16)

    s, x2, dv2_col = pl.pallas_call(
        _k3,
        grid=grid,
        in_specs=[row_spec(m), acc_spec(m, d), acc_spec(d, k),
                  acc_spec(1, k), row_spec(d)],
        out_specs=[row_spec(k), acc_spec(k, d), acc_spec(k, 1)],
        out_shape=[jax.ShapeDtypeStruct((n, k), bf16),
                   jax.ShapeDtypeStruct((k, d), f32),
                   jax.ShapeDtypeStruct((k, 1), f32)],
    )(hb, ep, Wp.T, bp[None, :], x)

    e2_raw = pl.pallas_call(
        _k4,
        grid=grid,
        in_specs=[row_spec(m), row_spec(k), acc_spec(k, d)],
        out_specs=acc_spec(m, d),
        out_shape=jax.ShapeDtypeStruct((m, d), f32),
    )(hb, s, x2)

    a = (_dot(e2_raw / de[:, None], W1.T)).astype(bf16)

    xn2 = pl.pallas_call(
        _k5,
        grid=grid,
        in_specs=[row_spec(m), row_spec(k), acc_spec(m, d)],
        out_specs=acc_spec(k, d),
        out_shape=jax.ShapeDtypeStruct((k, d), f32),
    )(hb, s, a)

    dv2 = jnp.clip(dv2_col[:, 0], 1e-6, None)
    x3 = jnp.maximum(xn2 / dv2[:, None] + b1[None, :], 0.0)
    return _dot(Wr, jnp.mean(x3, axis=0)) + br
